# baseline (speedup 1.0000x reference)
import functools

import jax
import jax.numpy as jnp
from jax import lax
from jax.experimental import pallas as pl
from jax.experimental.pallas import tpu as pltpu

_BATCH = 4096
_NB = 4
_RB = 128
_R = _NB * _RB
_NCH = _BATCH // _R


def _tc_body(idx_s, rot_any, tra_any, rot_o, tra_o, *scratch):
    rot_vs = scratch[0:_NB]
    tra_vs = scratch[_NB:2 * _NB]
    sems_r = scratch[2 * _NB:3 * _NB]
    sems_t = scratch[3 * _NB:4 * _NB]
    sem_w = scratch[4 * _NB]

    def chunk(c, _):
        base = c * _R

        for k in range(_NB):
            def fire(j, _):
                idx = idx_s[base + k * _RB + j]
                pltpu.make_async_copy(rot_any.at[idx], rot_vs[k].at[j],
                                      sems_r[k]).start()
                pltpu.make_async_copy(tra_any.at[idx], tra_vs[k].at[j],
                                      sems_t[k]).start()
                return ()

            lax.fori_loop(0, _RB, fire, ())

        for k in range(_NB):
            def drain(j, _):
                pltpu.make_async_copy(rot_any.at[0], rot_vs[k].at[j],
                                      sems_r[k]).wait()
                pltpu.make_async_copy(tra_any.at[0], tra_vs[k].at[j],
                                      sems_t[k]).wait()
                return ()

            lax.fori_loop(0, _RB, drain, ())

        for k in range(_NB):
            ob = base + k * _RB
            pltpu.make_async_copy(rot_vs[k], rot_o.at[pl.ds(ob, _RB)],
                                  sem_w).start()
            pltpu.make_async_copy(rot_vs[k], rot_o.at[pl.ds(ob, _RB)],
                                  sem_w).wait()
            pltpu.make_async_copy(tra_vs[k], tra_o.at[pl.ds(ob, _RB)],
                                  sem_w).start()
            pltpu.make_async_copy(tra_vs[k], tra_o.at[pl.ds(ob, _RB)],
                                  sem_w).wait()
        return ()

    lax.fori_loop(0, _NCH, chunk, ())


@jax.jit
def kernel(indexes, rotation_table, translation_table):
    grid_spec = pltpu.PrefetchScalarGridSpec(
        num_scalar_prefetch=1,
        grid=(1,),
        in_specs=[
            pl.BlockSpec(memory_space=pl.ANY),
            pl.BlockSpec(memory_space=pl.ANY),
        ],
        out_specs=[
            pl.BlockSpec(memory_space=pl.ANY),
            pl.BlockSpec(memory_space=pl.ANY),
        ],
        scratch_shapes=(
            [pltpu.VMEM((_RB, 6, 6), jnp.float32)] * _NB
            + [pltpu.VMEM((_RB, 6, 3), jnp.float32)] * _NB
            + [pltpu.SemaphoreType.DMA] * (2 * _NB)
            + [pltpu.SemaphoreType.DMA]
        ),
    )
    rot, tra = pl.pallas_call(
        _tc_body,
        grid_spec=grid_spec,
        out_shape=[
            jax.ShapeDtypeStruct((_BATCH, 6, 6), jnp.float32),
            jax.ShapeDtypeStruct((_BATCH, 6, 3), jnp.float32),
        ],
    )(indexes, rotation_table, translation_table)
    return (rot, tra)


# speedup vs baseline: 1.1623x; 1.1623x over previous
import functools

import jax
import jax.numpy as jnp
from jax import lax
from jax.experimental import pallas as pl
from jax.experimental.pallas import tpu as pltpu
from jax.experimental.pallas import tpu_sc as plsc

_BATCH = 4096

_INFO = plsc.get_sparse_core_info()
_NW = _INFO.num_cores * _INFO.num_subcores
_BPW = _BATCH // _NW
_CH = 32
_NCH = _BPW // _CH

_MESH = plsc.VectorSubcoreMesh(core_axis_name="c", subcore_axis_name="s")


@functools.partial(
    pl.kernel,
    mesh=_MESH,
    out_type=(
        jax.ShapeDtypeStruct((_BATCH, 6, 6), jnp.float32),
        jax.ShapeDtypeStruct((_BATCH, 6, 3), jnp.float32),
    ),
    scratch_types=[
        pltpu.VMEM((_BPW + 16,), jnp.int32),
        pltpu.VMEM((_CH, 6, 6), jnp.float32),
        pltpu.VMEM((_CH, 6, 3), jnp.float32),
        pltpu.SemaphoreType.DMA,
        pltpu.SemaphoreType.DMA,
    ],
)
def _gather_rows(idx_hbm, rot_hbm, tra_hbm, rot_out, tra_out,
                 idx_v, rot_v, tra_v, sem_r, sem_t):
    wid = lax.axis_index("s") * _INFO.num_cores + lax.axis_index("c")
    base = wid * _BPW
    pltpu.sync_copy(idx_hbm.at[pl.ds(base, _BPW)], idx_v.at[pl.ds(0, _BPW)])

    def chunk(c, _):
        cb = c * _CH

        def fire(i, _):
            idx = idx_v[pl.ds(cb + i, 16)][0]
            pltpu.async_copy(rot_hbm.at[idx], rot_v.at[i], sem_r)
            pltpu.async_copy(tra_hbm.at[idx], tra_v.at[i], sem_t)
            return ()

        lax.fori_loop(0, _CH, fire, ())

        def drain(i, _):
            pltpu.make_async_copy(rot_hbm.at[0], rot_v.at[i], sem_r).wait()
            pltpu.make_async_copy(tra_hbm.at[0], tra_v.at[i], sem_t).wait()
            return ()

        lax.fori_loop(0, _CH, drain, ())
        pltpu.sync_copy(rot_v, rot_out.at[pl.ds(base + cb, _CH)])
        pltpu.sync_copy(tra_v, tra_out.at[pl.ds(base + cb, _CH)])
        return ()

    lax.fori_loop(0, _NCH, chunk, ())


def kernel(indexes, rotation_table, translation_table):
    return _gather_rows(indexes, rotation_table, translation_table)
